# Initial kernel scaffold; baseline (speedup 1.0000x reference)
import jax
import jax.numpy as jnp
from jax.experimental import pallas as pl


def kernel(h, edge_index, edge_feat, W_node, W_edge, attention_src, attention_dst, W_msg):
    raise NotImplementedError("write your pallas kernel here")


# speedup vs baseline: 32.5222x; 32.5222x over previous
import functools

import jax
import jax.numpy as jnp
from jax import lax
from jax.experimental import pallas as pl
from jax.experimental.pallas import tpu as pltpu
from jax.experimental.pallas import tpu_sc as plsc

N = 10000
E = 320000
IN_FEATS = 128
OUT_FEATS = 16
EDGE_FEATS = 4
HEADS = 8
ROW = 144
NC, NS = 2, 16
NW = NC * NS
C = 80
EPT = E // NW
NCHUNK = EPT // C
NROWCHUNK = N // C


def _node_tables_body(h_ref, w1_ref, w2_ref, o1_ref, o2_ref):
    x = h_ref[...]
    o1_ref[...] = jnp.dot(x, w1_ref[...], preferred_element_type=jnp.float32)
    o2_ref[...] = jnp.dot(x, w2_ref[...], preferred_element_type=jnp.float32)


def _node_tables(h, w1, w2):
    bn = 1000
    return pl.pallas_call(
        _node_tables_body,
        grid=(N // bn,),
        in_specs=[
            pl.BlockSpec((bn, IN_FEATS), lambda i: (i, 0)),
            pl.BlockSpec((IN_FEATS, ROW), lambda i: (0, 0)),
            pl.BlockSpec((IN_FEATS, 16), lambda i: (0, 0)),
        ],
        out_specs=[
            pl.BlockSpec((bn, ROW), lambda i: (i, 0)),
            pl.BlockSpec((bn, 16), lambda i: (i, 0)),
        ],
        out_shape=[
            jax.ShapeDtypeStruct((N, ROW), jnp.float32),
            jax.ShapeDtypeStruct((N, 16), jnp.float32),
        ],
    )(h, w1, w2)


def _edge_table_body(ef_ref, w_ref, o_ref):
    o_ref[...] = jnp.dot(ef_ref[...], w_ref[...],
                         preferred_element_type=jnp.float32)


def _edge_table(edge_feat, w3):
    be = 8000
    return pl.pallas_call(
        _edge_table_body,
        grid=(E // be,),
        in_specs=[
            pl.BlockSpec((be, EDGE_FEATS), lambda i: (i, 0)),
            pl.BlockSpec((EDGE_FEATS, ROW), lambda i: (0, 0)),
        ],
        out_specs=pl.BlockSpec((be, ROW), lambda i: (i, 0)),
        out_shape=jax.ShapeDtypeStruct((E, ROW), jnp.float32),
    )(edge_feat, w3)


def _edge_pass_body(tsrc_hbm, tdst_hbm, src_hbm, dst_hbm, etab_hbm, out_hbm,
                    idx_v, srow_v, drow_v, etab_v, num_v, acc_sh):
    cid = lax.axis_index("c")
    sid = lax.axis_index("s")
    wid = sid * NC + cid

    @pl.loop(0, C)
    def _zrow(r):
        @pl.loop(0, ROW, step=16)
        def _zcol(k):
            num_v[r, pl.ds(k, 16)] = jnp.zeros((16,), jnp.float32)

    @pl.loop(sid, NROWCHUNK, step=NS)
    def _zacc(j):
        pltpu.sync_copy(num_v, acc_sh.at[pl.ds(j * C, C)])

    plsc.subcore_barrier()

    base0 = wid * EPT

    @pl.loop(0, NCHUNK)
    def _chunk(i):
        base = base0 + i * C
        pltpu.sync_copy(src_hbm.at[pl.ds(base, C)], idx_v.at[0])
        pltpu.sync_copy(dst_hbm.at[pl.ds(base, C)], idx_v.at[1])
        pltpu.sync_copy(tsrc_hbm.at[idx_v.at[0]], srow_v)
        pltpu.sync_copy(tdst_hbm.at[idx_v.at[1]], drow_v)
        pltpu.sync_copy(etab_hbm.at[pl.ds(base, C)], etab_v)

        @pl.loop(0, C)
        def _edge(c):
            a = (srow_v[c, pl.ds(128, 16)] + drow_v[c, :]
                 + etab_v[c, pl.ds(128, 16)])
            a = jnp.where(a >= 0.0, a, a * 0.2)
            p = jnp.exp(a)
            num_v[c, pl.ds(128, 16)] = p
            for k in range(HEADS):
                num_v[c, pl.ds(16 * k, 16)] = (
                    srow_v[c, pl.ds(16 * k, 16)]
                    + etab_v[c, pl.ds(16 * k, 16)]) * p[k]

        pltpu.sync_copy(num_v, acc_sh.at[idx_v.at[1]], add=True)

    plsc.subcore_barrier()

    @pl.loop(sid, NROWCHUNK, step=NS)
    def _wb(j):
        pltpu.sync_copy(acc_sh.at[pl.ds(j * C, C)],
                        out_hbm.at[cid, pl.ds(j * C, C)])


def _edge_pass(tsrc, tdst, src, dst, etab):
    mesh = plsc.VectorSubcoreMesh(core_axis_name="c", subcore_axis_name="s")
    f = pl.kernel(
        _edge_pass_body,
        out_type=jax.ShapeDtypeStruct((NC, N, ROW), jnp.float32),
        mesh=mesh,
        compiler_params=pltpu.CompilerParams(use_tc_tiling_on_sc=False),
        scratch_types=[
            pltpu.VMEM((2, C), jnp.int32),
            pltpu.VMEM((C, ROW), jnp.float32),
            pltpu.VMEM((C, 16), jnp.float32),
            pltpu.VMEM((C, ROW), jnp.float32),
            pltpu.VMEM((C, ROW), jnp.float32),
            pltpu.VMEM_SHARED((N, ROW), jnp.float32),
        ],
    )
    return f(tsrc, tdst, src, dst, etab)


def _finalize_body(acc_ref, o_ref):
    r = acc_ref[0] + acc_ref[1]
    acc = jnp.zeros(o_ref.shape, jnp.float32)
    for h in range(HEADS):
        den = jnp.maximum(r[:, 128 + h], 1e-12)
        acc = acc + r[:, 16 * h:16 * h + 16] / den[:, None]
    o_ref[...] = acc * (1.0 / HEADS)


def _finalize(parts):
    bn = 1000
    return pl.pallas_call(
        _finalize_body,
        grid=(N // bn,),
        in_specs=[pl.BlockSpec((NC, bn, ROW), lambda i: (0, i, 0))],
        out_specs=pl.BlockSpec((bn, 16), lambda i: (i, 0)),
        out_shape=jax.ShapeDtypeStruct((N, 16), jnp.float32),
    )(parts)


def kernel(h, edge_index, edge_feat, W_node, W_edge, attention_src,
           attention_dst, W_msg):
    f32 = jnp.float32
    wn3 = W_node.reshape(IN_FEATS, HEADS, OUT_FEATS)
    a_src = jnp.einsum("jhk,hk->jh", wn3, attention_src)
    a_dst = jnp.einsum("jhk,hk->jh", wn3, attention_dst)
    pad_n = jnp.zeros((IN_FEATS, 8), f32)
    pad_e = jnp.zeros((EDGE_FEATS, 8), f32)
    w1 = jnp.concatenate([W_msg[:IN_FEATS], a_src, pad_n], axis=1)
    w2 = jnp.concatenate([a_dst, pad_n], axis=1)
    w3 = jnp.concatenate([W_msg[IN_FEATS:], W_edge, pad_e], axis=1)

    tsrc, tdst = _node_tables(h, w1, w2)
    etab = _edge_table(edge_feat, w3)
    parts = _edge_pass(tsrc, tdst, edge_index[0], edge_index[1], etab)
    return _finalize(parts)
